# baseline (speedup 1.0000x reference)
import dataclasses

import jax
import jax.numpy as jnp
from jax import lax
from jax.experimental import pallas as pl
from jax.experimental.pallas import tpu as pltpu
from jax.experimental.pallas import tpu_sc as plsc

N = 10000
E = 320000
D_IN = 128
D_H = 128
D_OUT = 64
K = 10
ALPHA = 0.1

HALF = D_OUT // 2
NSUB = 16
NP = 10240
SROWS = NP // NSUB
CHUNK = 128
NRC = SROWS // CHUNK
EPS = E // NSUB
NCH = 160
EPAD = NCH * CHUNK - EPS
DUMP0 = 10200
NBUF = 4


def _mlp_body(feats_ref, w1_ref, b1_ref, w2_ref, b2_ref, h1_ref, h0s_ref):
    x = feats_ref[...]
    h = jnp.dot(x, w1_ref[...], preferred_element_type=jnp.float32) + b1_ref[...]
    h1_ref[...] = h
    h2 = (jnp.dot(jnp.maximum(h, 0.0), w2_ref[...],
                  preferred_element_type=jnp.float32) + b2_ref[...])
    h0s_ref[0, :, :] = h2[:, :HALF]
    h0s_ref[1, :, :] = h2[:, HALF:]


def _mlp(feats, w1, b1, w2, b2):
    blk = 1000
    grid = (N // blk,)
    return pl.pallas_call(
        _mlp_body,
        grid=grid,
        in_specs=[
            pl.BlockSpec((blk, D_IN), lambda i: (i, 0)),
            pl.BlockSpec((D_IN, D_H), lambda i: (0, 0)),
            pl.BlockSpec((1, D_H), lambda i: (0, 0)),
            pl.BlockSpec((D_H, D_OUT), lambda i: (0, 0)),
            pl.BlockSpec((1, D_OUT), lambda i: (0, 0)),
        ],
        out_specs=[
            pl.BlockSpec((blk, D_H), lambda i: (i, 0)),
            pl.BlockSpec((2, blk, HALF), lambda i: (0, i, 0)),
        ],
        out_shape=[
            jax.ShapeDtypeStruct((N, D_H), jnp.float32),
            jax.ShapeDtypeStruct((2, NP, HALF), jnp.float32),
        ],
    )(feats, w1, b1.reshape(1, D_H), w2, b2.reshape(1, D_OUT))


def _rsqrt16(dc):
    i = plsc.bitcast(dc, jnp.int32)
    i = jnp.int32(0x5F3759DF) - (i >> 1)
    y = plsc.bitcast(i, jnp.float32)
    for _ in range(3):
        y = y * (1.5 - 0.5 * dc * y * y)
    return y


def _appnp_body(src_hbm, dst_hbm, f0s_hbm, out_hbm,
                p_sp, acc_sp,
                srcx, dstx, g0, g1, g2, g3, zbuf, abuf, pbuf, dbuf, cvec,
                sg0, sg1, sg2, sg3, ss0, ss1, ss2, ss3, sz, sp0, sp1):
    c = lax.axis_index("c")
    s = lax.axis_index("s")
    nb = s * SROWS
    zero16 = jnp.zeros((16,), jnp.float32)
    one16 = jnp.full((16,), 1.0, jnp.float32)
    gb = (g0, g1, g2, g3)
    sg = (sg0, sg1, sg2, sg3)
    ss = (ss0, ss1, ss2, ss3)

    def zero_acc_async(r0):
        for q in range(4):
            pltpu.async_copy(zbuf, acc_sp.at[pl.ds(r0 + 32 * q, 32)], sz)

    def drain_zeros():
        @pl.loop(0, 4 * NRC)
        def _(_i):
            pltpu.make_async_copy(zbuf, acc_sp.at[pl.ds(nb, 32)], sz).wait()

    def wait_gather(buf, sem):
        pltpu.make_async_copy(p_sp.at[srcx.at[0]], buf, sem).wait()

    def wait_scatter(buf, sem):
        pltpu.make_async_copy(buf, acc_sp.at[dstx.at[0]], sem).wait()

    pltpu.sync_copy(src_hbm.at[s], srcx)
    pltpu.sync_copy(dst_hbm.at[s], dstx)

    @pl.loop(0, 32)
    def _(i):
        zbuf[i, pl.ds(0, 16)] = zero16
        zbuf[i, pl.ds(16, 16)] = zero16

    @pl.loop(0, CHUNK)
    def _(i):
        g3[i, pl.ds(0, 16)] = one16
        g3[i, pl.ds(16, 16)] = one16

    @pl.loop(0, NRC)
    def _(k):
        zero_acc_async(nb + k * CHUNK)

    drain_zeros()
    plsc.subcore_barrier()

    @pl.loop(0, NCH)
    def _(j):
        pltpu.async_copy(g3, acc_sp.at[dstx.at[j]], ss0, add=True)

    @pl.loop(0, NCH)
    def _(j):
        wait_scatter(g3, ss0)

    plsc.subcore_barrier()

    @pl.loop(0, NRC)
    def _(k):
        r0 = nb + k * CHUNK
        v0 = k * CHUNK
        pltpu.sync_copy(acc_sp.at[pl.ds(r0, CHUNK)], abuf)
        zero_acc_async(r0)
        pltpu.sync_copy(f0s_hbm.at[c, pl.ds(r0, CHUNK)], g0)

        @pl.loop(0, CHUNK)
        def _(i):
            dc = jnp.maximum(abuf[i, pl.ds(0, 16)], 1.0)
            dbuf[v0 + i, :] = dc
            y = _rsqrt16(dc)
            pa = y * g0[i, pl.ds(0, 16)]
            pb = y * g0[i, pl.ds(16, 16)]
            pbuf[i, pl.ds(0, 16)] = pa
            pbuf[i, pl.ds(16, 16)] = pb
            cvec[v0 + i, :] = plsc.pack(ALPHA * pa, ALPHA * pb,
                                        format=plsc.PackFormat.INTERLEAVED)

        pltpu.sync_copy(pbuf, p_sp.at[pl.ds(r0, CHUNK)])

    drain_zeros()
    plsc.subcore_barrier()

    @pl.loop(0, K)
    def _(t):
        for b in range(NBUF):
            pltpu.async_copy(p_sp.at[srcx.at[b]], gb[b], sg[b])

        @pl.loop(0, NCH, step=NBUF)
        def _(j):
            for b in range(NBUF):
                wait_gather(gb[b], sg[b])
                pltpu.async_copy(gb[b], acc_sp.at[dstx.at[j + b]],
                                 ss[b], add=True)
            for b in range(NBUF):
                wait_scatter(gb[b], ss[b])

                @pl.when(j + NBUF + b < NCH)
                def _(b=b):
                    pltpu.async_copy(p_sp.at[srcx.at[j + NBUF + b]],
                                     gb[b], sg[b])

        plsc.subcore_barrier()

        for k in range(NRC):
            r0 = nb + k * CHUNK
            v0 = k * CHUNK
            ob, spk = (pbuf, sp0) if k % 2 == 0 else (g0, sp1)
            pltpu.sync_copy(acc_sp.at[pl.ds(r0, CHUNK)], abuf)
            zero_acc_async(r0)
            if k >= 2:
                pltpu.make_async_copy(ob, p_sp.at[pl.ds(nb, CHUNK)],
                                      spk).wait()

            @pl.when(t < K - 1)
            def _(ob=ob, spk=spk, r0=r0, v0=v0):
                @pl.loop(0, CHUNK)
                def _(i):
                    a = (1.0 - ALPHA) / dbuf[v0 + i, :]
                    ca, cb = plsc.unpack(cvec[v0 + i, :],
                                         format=plsc.PackFormat.INTERLEAVED)
                    ob[i, pl.ds(0, 16)] = a * abuf[i, pl.ds(0, 16)] + ca
                    ob[i, pl.ds(16, 16)] = a * abuf[i, pl.ds(16, 16)] + cb

                pltpu.async_copy(ob, p_sp.at[pl.ds(r0, CHUNK)], spk)

            @pl.when(t == K - 1)
            def _(ob=ob, spk=spk, r0=r0, v0=v0):
                @pl.loop(0, CHUNK)
                def _(i):
                    dc = dbuf[v0 + i, :]
                    y = _rsqrt16(dc)
                    sq = dc * y
                    ay = (1.0 - ALPHA) * y
                    ca, cb = plsc.unpack(cvec[v0 + i, :],
                                         format=plsc.PackFormat.INTERLEAVED)
                    ob[i, pl.ds(0, 16)] = ay * abuf[i, pl.ds(0, 16)] + ca * sq
                    ob[i, pl.ds(16, 16)] = (ay * abuf[i, pl.ds(16, 16)]
                                            + cb * sq)

                pltpu.async_copy(ob, out_hbm.at[c, pl.ds(r0, CHUNK)], spk)

        pltpu.make_async_copy(g0, p_sp.at[pl.ds(nb, CHUNK)], sp1).wait()
        pltpu.make_async_copy(pbuf, p_sp.at[pl.ds(nb, CHUNK)], sp0).wait()
        drain_zeros()
        plsc.subcore_barrier()


_sc_params = pltpu.CompilerParams()
if "needs_layout_passes" in pltpu.CompilerParams.__dataclass_fields__:
    _sc_params = dataclasses.replace(_sc_params, needs_layout_passes=False)
if "use_tc_tiling_on_sc" in pltpu.CompilerParams.__dataclass_fields__:
    _sc_params = dataclasses.replace(_sc_params, use_tc_tiling_on_sc=False)

_appnp = pl.kernel(
    _appnp_body,
    out_type=jax.ShapeDtypeStruct((2, NP, HALF), jnp.float32),
    mesh=plsc.VectorSubcoreMesh(core_axis_name="c", subcore_axis_name="s"),
    compiler_params=_sc_params,
    scratch_types=[
        pltpu.VMEM_SHARED((NP, HALF), jnp.float32),
        pltpu.VMEM_SHARED((NP, HALF), jnp.float32),
        pltpu.VMEM((NCH, CHUNK), jnp.int32),
        pltpu.VMEM((NCH, CHUNK), jnp.int32),
        pltpu.VMEM((CHUNK, HALF), jnp.float32),
        pltpu.VMEM((CHUNK, HALF), jnp.float32),
        pltpu.VMEM((CHUNK, HALF), jnp.float32),
        pltpu.VMEM((CHUNK, HALF), jnp.float32),
        pltpu.VMEM((32, HALF), jnp.float32),
        pltpu.VMEM((CHUNK, HALF), jnp.float32),
        pltpu.VMEM((CHUNK, HALF), jnp.float32),
        pltpu.VMEM((SROWS, 16), jnp.float32),
        pltpu.VMEM((SROWS, HALF), jnp.bfloat16),
        pltpu.SemaphoreType.DMA,
        pltpu.SemaphoreType.DMA,
        pltpu.SemaphoreType.DMA,
        pltpu.SemaphoreType.DMA,
        pltpu.SemaphoreType.DMA,
        pltpu.SemaphoreType.DMA,
        pltpu.SemaphoreType.DMA,
        pltpu.SemaphoreType.DMA,
        pltpu.SemaphoreType.DMA,
        pltpu.SemaphoreType.DMA,
        pltpu.SemaphoreType.DMA,
    ],
)


def kernel(feats, edge_index, W1, b1, W2, b2):
    src = edge_index[0].reshape(NSUB, EPS)
    dst = edge_index[1].reshape(NSUB, EPS)
    pad_src = jnp.zeros((NSUB, EPAD), jnp.int32)
    pad_dst = jnp.broadcast_to(
        DUMP0 + jnp.arange(NSUB, dtype=jnp.int32)[:, None], (NSUB, EPAD))
    src = jnp.concatenate([src, pad_src], axis=1).reshape(NSUB, NCH, CHUNK)
    dst = jnp.concatenate([dst, pad_dst], axis=1).reshape(NSUB, NCH, CHUNK)

    h1, h0s = _mlp(feats, W1, b1, W2, b2)
    out = _appnp(src, dst, h0s)
    feat = jnp.concatenate([out[0, :N], out[1, :N]], axis=1)
    return (h1, feat)


# speedup vs baseline: 1.1835x; 1.1835x over previous
import dataclasses

import jax
import jax.numpy as jnp
from jax import lax
from jax.experimental import pallas as pl
from jax.experimental.pallas import tpu as pltpu
from jax.experimental.pallas import tpu_sc as plsc

N = 10000
E = 320000
D_IN = 128
D_H = 128
D_OUT = 64
K = 10
ALPHA = 0.1

HALF = D_OUT // 2
NSUB = 16
NP = 10240
SROWS = NP // NSUB
CHUNK = 128
NRC = SROWS // CHUNK
EPS = E // NSUB
NCH = 158
EPAD = NCH * CHUNK - EPS
DUMP0 = 10200


def _mlp_body(feats_ref, w1_ref, b1_ref, w2_ref, b2_ref, h1_ref, h0s_ref):
    x = feats_ref[...]
    h = jnp.dot(x, w1_ref[...], preferred_element_type=jnp.float32) + b1_ref[...]
    h1_ref[...] = h
    h2 = (jnp.dot(jnp.maximum(h, 0.0), w2_ref[...],
                  preferred_element_type=jnp.float32) + b2_ref[...])
    h0s_ref[0, :, :] = h2[:, :HALF]
    h0s_ref[1, :, :] = h2[:, HALF:]


def _mlp(feats, w1, b1, w2, b2):
    blk = 1000
    grid = (N // blk,)
    return pl.pallas_call(
        _mlp_body,
        grid=grid,
        in_specs=[
            pl.BlockSpec((blk, D_IN), lambda i: (i, 0)),
            pl.BlockSpec((D_IN, D_H), lambda i: (0, 0)),
            pl.BlockSpec((1, D_H), lambda i: (0, 0)),
            pl.BlockSpec((D_H, D_OUT), lambda i: (0, 0)),
            pl.BlockSpec((1, D_OUT), lambda i: (0, 0)),
        ],
        out_specs=[
            pl.BlockSpec((blk, D_H), lambda i: (i, 0)),
            pl.BlockSpec((2, blk, HALF), lambda i: (0, i, 0)),
        ],
        out_shape=[
            jax.ShapeDtypeStruct((N, D_H), jnp.float32),
            jax.ShapeDtypeStruct((2, NP, HALF), jnp.float32),
        ],
    )(feats, w1, b1.reshape(1, D_H), w2, b2.reshape(1, D_OUT))


def _rsqrt16(dc):
    i = plsc.bitcast(dc, jnp.int32)
    i = jnp.int32(0x5F3759DF) - (i >> 1)
    y = plsc.bitcast(i, jnp.float32)
    for _ in range(3):
        y = y * (1.5 - 0.5 * dc * y * y)
    return y


def _appnp_body(src_hbm, dst_hbm, f0s_hbm, out_hbm,
                p_sp, acc_sp,
                srcx, dstx, gbuf, gbuf2, zbuf, abuf, pbuf, dbuf, cvec,
                sg0, sg1, ss0, ss1, sz, sp0, sp1):
    c = lax.axis_index("c")
    s = lax.axis_index("s")
    nb = s * SROWS
    zero16 = jnp.zeros((16,), jnp.float32)
    one16 = jnp.full((16,), 1.0, jnp.float32)

    def zero_acc(r0):
        pltpu.sync_copy(zbuf, acc_sp.at[pl.ds(r0, 64)])
        pltpu.sync_copy(zbuf, acc_sp.at[pl.ds(r0 + 64, 64)])

    def zero_acc_async(r0):
        pltpu.async_copy(zbuf, acc_sp.at[pl.ds(r0, 64)], sz)
        pltpu.async_copy(zbuf, acc_sp.at[pl.ds(r0 + 64, 64)], sz)

    def drain_zeros():
        @pl.loop(0, 2 * NRC)
        def _(_i):
            pltpu.make_async_copy(zbuf, acc_sp.at[pl.ds(nb, 64)], sz).wait()

    def wait_gather(buf, sem):
        pltpu.make_async_copy(p_sp.at[srcx.at[0]], buf, sem).wait()

    def wait_scatter(buf, sem):
        pltpu.make_async_copy(buf, acc_sp.at[dstx.at[0]], sem).wait()

    pltpu.sync_copy(src_hbm.at[s], srcx)
    pltpu.sync_copy(dst_hbm.at[s], dstx)

    @pl.loop(0, 64)
    def _(i):
        zbuf[i, pl.ds(0, 16)] = zero16
        zbuf[i, pl.ds(16, 16)] = zero16

    @pl.loop(0, CHUNK)
    def _(i):
        pbuf[i, pl.ds(0, 16)] = one16
        pbuf[i, pl.ds(16, 16)] = one16

    @pl.loop(0, NRC)
    def _(k):
        zero_acc(nb + k * CHUNK)

    plsc.subcore_barrier()

    @pl.loop(0, NCH)
    def _(j):
        pltpu.async_copy(pbuf, acc_sp.at[dstx.at[j]], ss0, add=True)

    @pl.loop(0, NCH)
    def _(j):
        wait_scatter(pbuf, ss0)

    plsc.subcore_barrier()

    @pl.loop(0, NRC)
    def _(k):
        r0 = nb + k * CHUNK
        v0 = k * CHUNK
        pltpu.sync_copy(acc_sp.at[pl.ds(r0, CHUNK)], abuf)
        zero_acc(r0)
        pltpu.sync_copy(f0s_hbm.at[c, pl.ds(r0, CHUNK)], gbuf)

        @pl.loop(0, CHUNK)
        def _(i):
            dc = jnp.maximum(abuf[i, pl.ds(0, 16)], 1.0)
            dbuf[v0 + i, :] = dc
            y = _rsqrt16(dc)
            pa = y * gbuf[i, pl.ds(0, 16)]
            pb = y * gbuf[i, pl.ds(16, 16)]
            pbuf[i, pl.ds(0, 16)] = pa
            pbuf[i, pl.ds(16, 16)] = pb
            cvec[v0 + i, pl.ds(0, 16)] = ALPHA * pa
            cvec[v0 + i, pl.ds(16, 16)] = ALPHA * pb

        pltpu.sync_copy(pbuf, p_sp.at[pl.ds(r0, CHUNK)])

    plsc.subcore_barrier()

    @pl.loop(0, K)
    def _(t):
        pltpu.async_copy(p_sp.at[srcx.at[0]], gbuf, sg0)

        @pl.loop(0, NCH, step=2)
        def _(j):
            wait_gather(gbuf, sg0)
            pltpu.async_copy(gbuf, acc_sp.at[dstx.at[j]], ss0, add=True)

            @pl.when(j > 0)
            def _():
                wait_scatter(gbuf2, ss1)

            pltpu.async_copy(p_sp.at[srcx.at[j + 1]], gbuf2, sg1)
            wait_gather(gbuf2, sg1)
            pltpu.async_copy(gbuf2, acc_sp.at[dstx.at[j + 1]], ss1, add=True)
            wait_scatter(gbuf, ss0)

            @pl.when(j + 2 < NCH)
            def _():
                pltpu.async_copy(p_sp.at[srcx.at[j + 2]], gbuf, sg0)

        wait_scatter(gbuf2, ss1)
        plsc.subcore_barrier()

        for k in range(NRC):
            r0 = nb + k * CHUNK
            v0 = k * CHUNK
            ob, spk = (pbuf, sp0) if k % 2 == 0 else (gbuf, sp1)
            pltpu.sync_copy(acc_sp.at[pl.ds(r0, CHUNK)], abuf)
            zero_acc_async(r0)
            if k >= 2:
                pltpu.make_async_copy(ob, p_sp.at[pl.ds(nb, CHUNK)],
                                      spk).wait()

            @pl.when(t < K - 1)
            def _(ob=ob, spk=spk, r0=r0, v0=v0):
                @pl.loop(0, CHUNK)
                def _(i):
                    a = (1.0 - ALPHA) / dbuf[v0 + i, :]
                    ob[i, pl.ds(0, 16)] = (a * abuf[i, pl.ds(0, 16)]
                                           + cvec[v0 + i, pl.ds(0, 16)])
                    ob[i, pl.ds(16, 16)] = (a * abuf[i, pl.ds(16, 16)]
                                            + cvec[v0 + i, pl.ds(16, 16)])

                pltpu.async_copy(ob, p_sp.at[pl.ds(r0, CHUNK)], spk)

            @pl.when(t == K - 1)
            def _(ob=ob, spk=spk, r0=r0, v0=v0):
                @pl.loop(0, CHUNK)
                def _(i):
                    dc = dbuf[v0 + i, :]
                    y = _rsqrt16(dc)
                    sq = dc * y
                    ay = (1.0 - ALPHA) * y
                    ob[i, pl.ds(0, 16)] = (
                        ay * abuf[i, pl.ds(0, 16)]
                        + cvec[v0 + i, pl.ds(0, 16)] * sq)
                    ob[i, pl.ds(16, 16)] = (
                        ay * abuf[i, pl.ds(16, 16)]
                        + cvec[v0 + i, pl.ds(16, 16)] * sq)

                pltpu.async_copy(ob, out_hbm.at[c, pl.ds(r0, CHUNK)], spk)

        pltpu.make_async_copy(gbuf, p_sp.at[pl.ds(nb, CHUNK)], sp1).wait()
        pltpu.make_async_copy(pbuf, p_sp.at[pl.ds(nb, CHUNK)], sp0).wait()
        drain_zeros()
        plsc.subcore_barrier()


_sc_params = pltpu.CompilerParams()
if "needs_layout_passes" in pltpu.CompilerParams.__dataclass_fields__:
    _sc_params = dataclasses.replace(_sc_params, needs_layout_passes=False)
if "use_tc_tiling_on_sc" in pltpu.CompilerParams.__dataclass_fields__:
    _sc_params = dataclasses.replace(_sc_params, use_tc_tiling_on_sc=False)

_appnp = pl.kernel(
    _appnp_body,
    out_type=jax.ShapeDtypeStruct((2, NP, HALF), jnp.float32),
    mesh=plsc.VectorSubcoreMesh(core_axis_name="c", subcore_axis_name="s"),
    compiler_params=_sc_params,
    scratch_types=[
        pltpu.VMEM_SHARED((NP, HALF), jnp.float32),
        pltpu.VMEM_SHARED((NP, HALF), jnp.float32),
        pltpu.VMEM((NCH, CHUNK), jnp.int32),
        pltpu.VMEM((NCH, CHUNK), jnp.int32),
        pltpu.VMEM((CHUNK, HALF), jnp.float32),
        pltpu.VMEM((CHUNK, HALF), jnp.float32),
        pltpu.VMEM((64, HALF), jnp.float32),
        pltpu.VMEM((CHUNK, HALF), jnp.float32),
        pltpu.VMEM((CHUNK, HALF), jnp.float32),
        pltpu.VMEM((SROWS, 16), jnp.float32),
        pltpu.VMEM((SROWS, HALF), jnp.float32),
        pltpu.SemaphoreType.DMA,
        pltpu.SemaphoreType.DMA,
        pltpu.SemaphoreType.DMA,
        pltpu.SemaphoreType.DMA,
        pltpu.SemaphoreType.DMA,
        pltpu.SemaphoreType.DMA,
        pltpu.SemaphoreType.DMA,
    ],
)


def kernel(feats, edge_index, W1, b1, W2, b2):
    src = edge_index[0].reshape(NSUB, EPS)
    dst = edge_index[1].reshape(NSUB, EPS)
    pad_src = jnp.zeros((NSUB, EPAD), jnp.int32)
    pad_dst = jnp.broadcast_to(
        DUMP0 + jnp.arange(NSUB, dtype=jnp.int32)[:, None], (NSUB, EPAD))
    src = jnp.concatenate([src, pad_src], axis=1).reshape(NSUB, NCH, CHUNK)
    dst = jnp.concatenate([dst, pad_dst], axis=1).reshape(NSUB, NCH, CHUNK)

    h1, h0s = _mlp(feats, W1, b1, W2, b2)
    out = _appnp(src, dst, h0s)
    feat = jnp.concatenate([out[0, :N], out[1, :N]], axis=1)
    return (h1, feat)
